# baseline (speedup 1.0000x reference)
import functools
from typing import Any

import jax
import jax.numpy as jnp
import numpy as np
from jax import lax
from jax.experimental import pallas as pl
from jax.experimental.pallas import tpu as pltpu
from jax.experimental.pallas import tpu_sc as plsc


def _ew_body(ef_ref, w_ref, b_ref, out_ref):
    D = ef_ref.shape[0]
    acc = jnp.full((1, ef_ref.shape[1]), b_ref[0, 0], jnp.float32)
    for j in range(D):
        acc = acc + w_ref[j, 0] * ef_ref[j:j + 1, :]
    out_ref[...] = jnp.maximum(acc, 0.0)


def _edge_weights(ef, W_el, b_el):
    E = ef.shape[0]
    D = ef.shape[1]
    efT = jnp.transpose(ef)
    Be = 32000
    grid = (E // Be,)
    out = pl.pallas_call(
        _ew_body,
        grid=grid,
        in_specs=[
            pl.BlockSpec((D, Be), lambda i: (0, i)),
            pl.BlockSpec((D, 1), lambda i: (0, 0)),
            pl.BlockSpec((1, 1), lambda i: (0, 0)),
        ],
        out_specs=pl.BlockSpec((1, Be), lambda i: (0, i)),
        out_shape=jax.ShapeDtypeStruct((1, E), jnp.float32),
    )(efT, W_el, b_el.reshape(1, 1))
    return out[0]


def _proj_body(x_ref, w_ref, b_ref, deg_ref, out_ref):
    dinv = lax.rsqrt(deg_ref[...])
    out_ref[0] = dinv * (
        jnp.dot(x_ref[0], w_ref[...], preferred_element_type=jnp.float32)
        + b_ref[...]
    )


def _project(xT, W_comb, b_comb, degp1, bn):
    T, N, F = xT.shape
    K = W_comb.shape[1]
    grid = (T, N // bn)
    return pl.pallas_call(
        _proj_body,
        grid=grid,
        in_specs=[
            pl.BlockSpec((1, bn, F), lambda t, i: (t, i, 0)),
            pl.BlockSpec((F, K), lambda t, i: (0, 0)),
            pl.BlockSpec((1, K), lambda t, i: (0, 0)),
            pl.BlockSpec((bn, 1), lambda t, i: (i, 0)),
        ],
        out_specs=pl.BlockSpec((1, bn, K), lambda t, i: (t, i, 0)),
        out_shape=jax.ShapeDtypeStruct((T, N, K), jnp.float32),
    )(xT, W_comb, b_comb.reshape(1, K), degp1)


def _recur_body(vp_ref, u_ref, deg_ref, bpost_ref,
                wze_ref, wre_ref, whe_ref,
                wlz2_ref, wlr2_ref, wlh2_ref,
                blz_ref, blr_ref, blh_ref,
                probs_ref, wout_ref, bout_ref, out_ref):
    T = u_ref.shape[0]
    bn = u_ref.shape[1]
    HID = wlz2_ref.shape[0]
    dinv = lax.rsqrt(deg_ref[...])
    H = jnp.zeros((bn, HID), jnp.float32)
    Hacc = jnp.zeros((bn, HID), jnp.float32)
    for t in range(T):
        G = dinv * (vp_ref[t] + u_ref[t]) + bpost_ref[...]
        Z = jax.nn.sigmoid(
            jnp.dot(G, wze_ref[...], preferred_element_type=jnp.float32)
            + jnp.dot(H, wlz2_ref[...], preferred_element_type=jnp.float32)
            + blz_ref[...])
        R = jax.nn.sigmoid(
            jnp.dot(G, wre_ref[...], preferred_element_type=jnp.float32)
            + jnp.dot(H, wlr2_ref[...], preferred_element_type=jnp.float32)
            + blr_ref[...])
        Ht = jnp.tanh(
            jnp.dot(G, whe_ref[...], preferred_element_type=jnp.float32)
            + jnp.dot(H * R, wlh2_ref[...], preferred_element_type=jnp.float32)
            + blh_ref[...])
        H = Z * H + (1.0 - Z) * Ht
        Hacc = Hacc + probs_ref[0, t] * H
    out_ref[...] = (
        jnp.dot(jnp.maximum(Hacc, 0.0), wout_ref[...],
                preferred_element_type=jnp.float32)
        + bout_ref[...])


def _recurrence(Vp, U, degp1, bpost, Wze, Wre, Whe, Wlz2, Wlr2, Wlh2,
                blz, blr, blh, probs, W_out, b_out, bn):
    T, N, K = U.shape
    HID = Wlz2.shape[0]
    TO = W_out.shape[1]
    grid = (N // bn,)
    full = lambda shape: pl.BlockSpec(shape, lambda i: tuple(0 for _ in shape))
    return pl.pallas_call(
        _recur_body,
        grid=grid,
        in_specs=[
            pl.BlockSpec((T, bn, K), lambda i: (0, i, 0)),
            pl.BlockSpec((T, bn, K), lambda i: (0, i, 0)),
            pl.BlockSpec((bn, 1), lambda i: (i, 0)),
            full((1, K)),
            full((K, HID)), full((K, HID)), full((K, HID)),
            full((HID, HID)), full((HID, HID)), full((HID, HID)),
            full((1, HID)), full((1, HID)), full((1, HID)),
            full((1, T)), full((HID, TO)), full((1, TO)),
        ],
        out_specs=pl.BlockSpec((bn, TO), lambda i: (i, 0)),
        out_shape=jax.ShapeDtypeStruct((N, TO), jnp.float32),
    )(Vp, U, degp1, bpost.reshape(1, K),
      Wze, Wre, Whe, Wlz2, Wlr2, Wlh2,
      blz.reshape(1, HID), blr.reshape(1, HID), blh.reshape(1, HID),
      probs.reshape(1, T), W_out, b_out.reshape(1, TO))


_NT = 16
_NW = 32
_CW = 128
_L = 16


def _sc_mesh():
    return plsc.VectorSubcoreMesh(
        core_axis_name="c", subcore_axis_name="s",
        num_cores=2, num_subcores=_NT)


def _sc_degree(col32, ew32, npad):
    nch = col32.shape[1]
    rpt = npad // _NT

    @functools.partial(
        pl.kernel, mesh=_sc_mesh(),
        compiler_params=pltpu.CompilerParams(needs_layout_passes=False),
        out_type=jax.ShapeDtypeStruct((2, npad), jnp.float32),
        scratch_types=[
            pltpu.VMEM_SHARED((npad,), jnp.float32),
            pltpu.VMEM((_CW,), jnp.int32),
            pltpu.VMEM((_CW,), jnp.float32),
            pltpu.VMEM((rpt,), jnp.float32),
        ],
    )
    def deg_kernel(col_hbm, ew_hbm, deg_hbm, sh_deg, cbuf, wbuf, zbuf):
        cid = lax.axis_index("c")
        tid = lax.axis_index("s")
        base = tid * rpt
        zero16 = jnp.zeros((_L,), jnp.float32)

        def zd(i, _):
            zbuf[pl.ds(i * _L, _L)] = zero16
            return 0
        lax.fori_loop(0, rpt // _L, zd, 0)
        pltpu.sync_copy(zbuf, sh_deg.at[pl.ds(base, rpt)])
        plsc.subcore_barrier()

        slab = cid * _NT + tid

        def chunk(j, _):
            pltpu.sync_copy(col_hbm.at[slab, j], cbuf)
            pltpu.sync_copy(ew_hbm.at[slab, j], wbuf)
            pltpu.sync_copy(wbuf, sh_deg.at[cbuf], add=True)
            return 0
        lax.fori_loop(0, nch, chunk, 0)
        plsc.subcore_barrier()

        pltpu.sync_copy(sh_deg.at[pl.ds(base, rpt)],
                        deg_hbm.at[cid, pl.ds(base, rpt)])

    return deg_kernel(col32, ew32)


def _sc_segment(U2, pk, T, N, K, npad, t_per_core):
    nc = pk.shape[1]
    scw = pk.shape[3]
    rpt = npad // _NT
    assert nc % 3 == 0

    @functools.partial(
        pl.kernel, mesh=_sc_mesh(),
        compiler_params=pltpu.CompilerParams(
            needs_layout_passes=False, use_tc_tiling_on_sc=False),
        out_type=jax.ShapeDtypeStruct((T * npad, K), jnp.float32),
        scratch_types=[
            pltpu.VMEM_SHARED((npad, K), jnp.float32),
            pltpu.VMEM((3, scw), jnp.int32),
            pltpu.VMEM((3, scw), jnp.int32),
            pltpu.VMEM((3, scw), jnp.int32),
            pltpu.VMEM((scw,), jnp.int32),
            pltpu.VMEM((scw,), jnp.int32),
            pltpu.VMEM((scw,), jnp.int32),
            pltpu.VMEM((scw,), jnp.int32),
            pltpu.VMEM((scw,), jnp.int32),
            pltpu.VMEM((scw,), jnp.int32),
            pltpu.VMEM((scw, K), jnp.float32),
            pltpu.VMEM((scw, K), jnp.float32),
            pltpu.VMEM((scw, K), jnp.float32),
            pltpu.VMEM((_L, K), jnp.float32),
            pltpu.SemaphoreType.DMA, pltpu.SemaphoreType.DMA,
            pltpu.SemaphoreType.DMA, pltpu.SemaphoreType.DMA,
            pltpu.SemaphoreType.DMA, pltpu.SemaphoreType.DMA,
            pltpu.SemaphoreType.DMA, pltpu.SemaphoreType.DMA,
            pltpu.SemaphoreType.DMA,
        ],
    )
    def seg_kernel(pk_hbm, u_hbm, vp_hbm, sh_v,
                   ebuf0, ebuf1, ebuf2, cbuf0, cbuf1, cbuf2,
                   rowt0, rowt1, rowt2, gbuf0, gbuf1, gbuf2, zbuf,
                   se0, se1, se2, sg0, sg1, sg2, ss0, ss1, ss2):
        cid = lax.axis_index("c")
        tid = lax.axis_index("s")
        base = tid * rpt
        zero16 = jnp.zeros((_L,), jnp.float32)
        gbuf = (gbuf0, gbuf1, gbuf2)
        ebuf = (ebuf0, ebuf1, ebuf2)
        cbuf = (cbuf0, cbuf1, cbuf2)
        rowt = (rowt0, rowt1, rowt2)
        se = (se0, se1, se2)
        sg = (sg0, sg1, sg2)
        ss = (ss0, ss1, ss2)

        def zb(i, _):
            for c in range(K // _L):
                zbuf[i, pl.ds(c * _L, _L)] = zero16
            return 0
        lax.fori_loop(0, _L, zb, 0)

        def fill_rowt(p, tbase):
            for k in range(scw // _L):
                rowt[p][pl.ds(k * _L, _L)] = (
                    ebuf[p][0, pl.ds(k * _L, _L)] + tbase)

        def scale_and_cbuf(p):
            def scale(g, _):
                wv = plsc.bitcast(ebuf[p][2, pl.ds(g * _L, _L)], jnp.float32)
                cbuf[p][pl.ds(g * _L, _L)] = ebuf[p][1, pl.ds(g * _L, _L)]
                for l in range(_L):
                    s = wv[l]
                    e = g * _L + l
                    for c in range(K // _L):
                        gbuf[p][e, pl.ds(c * _L, _L)] = (
                            s * gbuf[p][e, pl.ds(c * _L, _L)])
                return 0
            lax.fori_loop(0, scw // _L, scale, 0)

        def eload(p, m):
            pltpu.async_copy(pk_hbm.at[tid, m], ebuf[p], se[p])

        def ewait(p, m):
            pltpu.make_async_copy(pk_hbm.at[tid, m], ebuf[p], se[p]).wait()

        def gstart(p):
            pltpu.async_copy(u_hbm.at[rowt[p]], gbuf[p], sg[p])

        def gwait(p):
            pltpu.make_async_copy(u_hbm.at[rowt[p]], gbuf[p], sg[p]).wait()

        def sstart(p):
            pltpu.async_copy(gbuf[p], sh_v.at[cbuf[p]], ss[p], add=True)

        def swait(p):
            pltpu.make_async_copy(gbuf[p], sh_v.at[cbuf[p]], ss[p]).wait()

        def per_t(tt, _):
            t = cid * t_per_core + tt
            tbase = t * N

            def zv(z, _):
                pltpu.sync_copy(zbuf, sh_v.at[pl.ds(base + z * _L, _L)])
                return 0
            lax.fori_loop(0, rpt // _L, zv, 0)
            plsc.subcore_barrier()

            eload(0, 0)
            eload(1, 1)
            eload(2, 2)
            ewait(0, 0)
            fill_rowt(0, tbase)
            gstart(0)
            ewait(1, 1)
            fill_rowt(1, tbase)
            gstart(1)

            def triple(m, _):
                not_last = m < nc // 3 - 1
                for r in range(3):
                    c3 = 3 * m + r
                    p = r
                    q2 = (r + 2) % 3

                    def prefetch():
                        ewait(q2, c3 + 2)
                        fill_rowt(q2, tbase)

                        @pl.when((m > 0) | (r > 0))
                        def _():
                            swait(q2)
                        gstart(q2)

                    if r == 0:
                        prefetch()
                    else:
                        @pl.when(not_last)
                        def _():
                            prefetch()

                    gwait(p)
                    scale_and_cbuf(p)
                    sstart(p)

                    @pl.when(not_last)
                    def _():
                        eload(p, c3 + 3)
                return 0
            lax.fori_loop(0, nc // 3, triple, 0)
            swait(0)
            swait(1)
            swait(2)
            plsc.subcore_barrier()

            pltpu.sync_copy(sh_v.at[pl.ds(base, rpt)],
                            vp_hbm.at[pl.ds(t * npad + base, rpt)])
            plsc.subcore_barrier()
            return 0
        lax.fori_loop(0, t_per_core, per_t, 0)

    return seg_kernel(pk, U2)


def kernel(x, edge_index, edge_features, W_fl, b_fl, W_el, b_el, att,
           W_cz, b_cz, W_cr, b_cr, W_ch, b_ch, W_lz, b_lz, W_lr, b_lr,
           W_lh, b_lh, W_out, b_out):
    N, F, T = x.shape
    HID = W_cz.shape[1]
    K = 3 * HID

    W_all = jnp.concatenate([W_cz, W_cr, W_ch], axis=1)
    W_comb = W_fl @ W_all
    b_comb = b_fl @ W_all
    bpost = jnp.concatenate([b_cz, b_cr, b_ch])
    Wlz1, Wlz2 = W_lz[:HID], W_lz[HID:]
    Wlr1, Wlr2 = W_lr[:HID], W_lr[HID:]
    Wlh1, Wlh2 = W_lh[:HID], W_lh[HID:]
    z = jnp.zeros((HID, HID), jnp.float32)
    Wze = jnp.concatenate([Wlz1, z, z], axis=0)
    Wre = jnp.concatenate([z, Wlr1, z], axis=0)
    Whe = jnp.concatenate([z, z, Wlh1], axis=0)
    probs = jax.nn.softmax(att)

    row = edge_index[0]
    col = edge_index[1]

    ew = _edge_weights(edge_features, W_el, b_el)

    E = row.shape[0]
    nch = -(-E // (_NW * _CW))
    epad = _NW * nch * _CW - E
    npad = ((N + _NT * _L - 1) // (_NT * _L)) * (_NT * _L)
    zi = jnp.zeros((epad,), jnp.int32)
    row32 = jnp.concatenate([row, zi]).reshape(_NW, nch, _CW)
    col32 = jnp.concatenate([col, zi]).reshape(_NW, nch, _CW)
    ew32 = jnp.concatenate([ew, jnp.zeros((epad,), jnp.float32)]).reshape(
        _NW, nch, _CW)

    deg2 = _sc_degree(col32, ew32, npad)
    degp1 = (deg2[0, :N] + deg2[1, :N] + 1.0).reshape(N, 1)

    xT = jnp.transpose(x, (2, 0, 1))
    bn = 1000 if N % 1000 == 0 else N
    U = _project(xT, W_comb, b_comb, degp1, bn)

    U2 = U.reshape(T * N, K)
    scw = 128
    nc = -(-E // (_NT * scw))
    nc = ((nc + 2) // 3) * 3
    epad2 = _NT * nc * scw - E
    zi2 = jnp.zeros((epad2,), jnp.int32)
    row16 = jnp.concatenate([row, zi2]).reshape(_NT, nc, scw)
    col16 = jnp.concatenate([col, zi2]).reshape(_NT, nc, scw)
    ewb16 = lax.bitcast_convert_type(
        jnp.concatenate([ew, jnp.zeros((epad2,), jnp.float32)]),
        jnp.int32).reshape(_NT, nc, scw)
    pk = jnp.stack([row16, col16, ewb16], axis=2)
    Vp_flat = _sc_segment(U2, pk, T, N, K, npad, T // 2)
    Vp = Vp_flat.reshape(T, npad, K)

    return _recurrence(Vp, U, degp1, bpost, Wze, Wre, Whe,
                       Wlz2, Wlr2, Wlh2, b_lz, b_lr, b_lh,
                       probs, W_out, b_out, bn)


# speedup vs baseline: 1.7571x; 1.7571x over previous
import functools
from typing import Any

import jax
import jax.numpy as jnp
import numpy as np
from jax import lax
from jax.experimental import pallas as pl
from jax.experimental.pallas import tpu as pltpu
from jax.experimental.pallas import tpu_sc as plsc


def _ew_body(ef_ref, w_ref, b_ref, out_ref):
    D = ef_ref.shape[0]
    acc = jnp.full((1, ef_ref.shape[1]), b_ref[0, 0], jnp.float32)
    for j in range(D):
        acc = acc + w_ref[j, 0] * ef_ref[j:j + 1, :]
    out_ref[...] = jnp.maximum(acc, 0.0)


def _edge_weights(ef, W_el, b_el):
    E = ef.shape[0]
    D = ef.shape[1]
    efT = jnp.transpose(ef)
    Be = 32000
    grid = (E // Be,)
    out = pl.pallas_call(
        _ew_body,
        grid=grid,
        in_specs=[
            pl.BlockSpec((D, Be), lambda i: (0, i)),
            pl.BlockSpec((D, 1), lambda i: (0, 0)),
            pl.BlockSpec((1, 1), lambda i: (0, 0)),
        ],
        out_specs=pl.BlockSpec((1, Be), lambda i: (0, i)),
        out_shape=jax.ShapeDtypeStruct((1, E), jnp.float32),
    )(efT, W_el, b_el.reshape(1, 1))
    return out[0]


def _proj_body(x_ref, w_ref, b_ref, deg_ref, out_ref):
    dinv = lax.rsqrt(deg_ref[...])
    out_ref[0] = dinv * (
        jnp.dot(x_ref[0], w_ref[...], preferred_element_type=jnp.float32)
        + b_ref[...]
    )


def _project(xT, W_comb, b_comb, degp1, bn):
    T, N, F = xT.shape
    K = W_comb.shape[1]
    grid = (T, N // bn)
    return pl.pallas_call(
        _proj_body,
        grid=grid,
        in_specs=[
            pl.BlockSpec((1, bn, F), lambda t, i: (t, i, 0)),
            pl.BlockSpec((F, K), lambda t, i: (0, 0)),
            pl.BlockSpec((1, K), lambda t, i: (0, 0)),
            pl.BlockSpec((bn, 1), lambda t, i: (i, 0)),
        ],
        out_specs=pl.BlockSpec((1, bn, K), lambda t, i: (t, i, 0)),
        out_shape=jax.ShapeDtypeStruct((T, N, K), jnp.float32),
    )(xT, W_comb, b_comb.reshape(1, K), degp1)


def _recur_body(vp_ref, u_ref, deg_ref, bpost_ref,
                wze_ref, wre_ref, whe_ref,
                wlz2_ref, wlr2_ref, wlh2_ref,
                blz_ref, blr_ref, blh_ref,
                probs_ref, wout_ref, bout_ref, out_ref):
    T = u_ref.shape[0]
    bn = u_ref.shape[1]
    HID = wlz2_ref.shape[0]
    dinv = lax.rsqrt(deg_ref[...])
    H = jnp.zeros((bn, HID), jnp.float32)
    Hacc = jnp.zeros((bn, HID), jnp.float32)
    for t in range(T):
        G = dinv * (vp_ref[t] + u_ref[t]) + bpost_ref[...]
        Z = jax.nn.sigmoid(
            jnp.dot(G, wze_ref[...], preferred_element_type=jnp.float32)
            + jnp.dot(H, wlz2_ref[...], preferred_element_type=jnp.float32)
            + blz_ref[...])
        R = jax.nn.sigmoid(
            jnp.dot(G, wre_ref[...], preferred_element_type=jnp.float32)
            + jnp.dot(H, wlr2_ref[...], preferred_element_type=jnp.float32)
            + blr_ref[...])
        Ht = jnp.tanh(
            jnp.dot(G, whe_ref[...], preferred_element_type=jnp.float32)
            + jnp.dot(H * R, wlh2_ref[...], preferred_element_type=jnp.float32)
            + blh_ref[...])
        H = Z * H + (1.0 - Z) * Ht
        Hacc = Hacc + probs_ref[0, t] * H
    out_ref[...] = (
        jnp.dot(jnp.maximum(Hacc, 0.0), wout_ref[...],
                preferred_element_type=jnp.float32)
        + bout_ref[...])


def _recurrence(Vp, U, degp1, bpost, Wze, Wre, Whe, Wlz2, Wlr2, Wlh2,
                blz, blr, blh, probs, W_out, b_out, bn):
    T, N, K = U.shape
    HID = Wlz2.shape[0]
    TO = W_out.shape[1]
    grid = (N // bn,)
    full = lambda shape: pl.BlockSpec(shape, lambda i: tuple(0 for _ in shape))
    return pl.pallas_call(
        _recur_body,
        grid=grid,
        in_specs=[
            pl.BlockSpec((T, bn, K), lambda i: (0, i, 0)),
            pl.BlockSpec((T, bn, K), lambda i: (0, i, 0)),
            pl.BlockSpec((bn, 1), lambda i: (i, 0)),
            full((1, K)),
            full((K, HID)), full((K, HID)), full((K, HID)),
            full((HID, HID)), full((HID, HID)), full((HID, HID)),
            full((1, HID)), full((1, HID)), full((1, HID)),
            full((1, T)), full((HID, TO)), full((1, TO)),
        ],
        out_specs=pl.BlockSpec((bn, TO), lambda i: (i, 0)),
        out_shape=jax.ShapeDtypeStruct((N, TO), jnp.float32),
    )(Vp, U, degp1, bpost.reshape(1, K),
      Wze, Wre, Whe, Wlz2, Wlr2, Wlh2,
      blz.reshape(1, HID), blr.reshape(1, HID), blh.reshape(1, HID),
      probs.reshape(1, T), W_out, b_out.reshape(1, TO))


_NT = 16
_NW = 32
_CW = 128
_L = 16


def _sc_mesh():
    return plsc.VectorSubcoreMesh(
        core_axis_name="c", subcore_axis_name="s",
        num_cores=2, num_subcores=_NT)


def _sc_degree(col32, ew32, npad):
    nch = col32.shape[1]
    rpt = npad // _NT

    @functools.partial(
        pl.kernel, mesh=_sc_mesh(),
        compiler_params=pltpu.CompilerParams(needs_layout_passes=False),
        out_type=jax.ShapeDtypeStruct((2, npad), jnp.float32),
        scratch_types=[
            pltpu.VMEM_SHARED((npad,), jnp.float32),
            pltpu.VMEM((_CW,), jnp.int32),
            pltpu.VMEM((_CW,), jnp.float32),
            pltpu.VMEM((rpt,), jnp.float32),
        ],
    )
    def deg_kernel(col_hbm, ew_hbm, deg_hbm, sh_deg, cbuf, wbuf, zbuf):
        cid = lax.axis_index("c")
        tid = lax.axis_index("s")
        base = tid * rpt
        zero16 = jnp.zeros((_L,), jnp.float32)

        def zd(i, _):
            zbuf[pl.ds(i * _L, _L)] = zero16
            return 0
        lax.fori_loop(0, rpt // _L, zd, 0)
        pltpu.sync_copy(zbuf, sh_deg.at[pl.ds(base, rpt)])
        plsc.subcore_barrier()

        slab = cid * _NT + tid

        def chunk(j, _):
            pltpu.sync_copy(col_hbm.at[slab, j], cbuf)
            pltpu.sync_copy(ew_hbm.at[slab, j], wbuf)
            pltpu.sync_copy(wbuf, sh_deg.at[cbuf], add=True)
            return 0
        lax.fori_loop(0, nch, chunk, 0)
        plsc.subcore_barrier()

        pltpu.sync_copy(sh_deg.at[pl.ds(base, rpt)],
                        deg_hbm.at[cid, pl.ds(base, rpt)])

    return deg_kernel(col32, ew32)


def _sc_segment(U2, pk, T, N, K, npad, t_per_core):
    nc = pk.shape[1]
    scw = pk.shape[3]
    rpt = npad // _NT
    assert nc % 3 == 0

    @functools.partial(
        pl.kernel, mesh=_sc_mesh(),
        compiler_params=pltpu.CompilerParams(needs_layout_passes=False),
        out_type=jax.ShapeDtypeStruct((T * npad, K), jnp.float32),
        scratch_types=[
            pltpu.VMEM_SHARED((npad, K), jnp.float32),
            pltpu.VMEM((3, scw), jnp.int32),
            pltpu.VMEM((3, scw), jnp.int32),
            pltpu.VMEM((3, scw), jnp.int32),
            pltpu.VMEM((scw,), jnp.int32),
            pltpu.VMEM((scw,), jnp.int32),
            pltpu.VMEM((scw,), jnp.int32),
            pltpu.VMEM((scw,), jnp.int32),
            pltpu.VMEM((scw,), jnp.int32),
            pltpu.VMEM((scw,), jnp.int32),
            pltpu.VMEM((scw, K), jnp.float32),
            pltpu.VMEM((scw, K), jnp.float32),
            pltpu.VMEM((scw, K), jnp.float32),
            pltpu.VMEM((_L, K), jnp.float32),
            pltpu.SemaphoreType.DMA, pltpu.SemaphoreType.DMA,
            pltpu.SemaphoreType.DMA, pltpu.SemaphoreType.DMA,
            pltpu.SemaphoreType.DMA, pltpu.SemaphoreType.DMA,
            pltpu.SemaphoreType.DMA, pltpu.SemaphoreType.DMA,
            pltpu.SemaphoreType.DMA,
        ],
    )
    def seg_kernel(pk_hbm, u_hbm, vp_hbm, sh_v,
                   ebuf0, ebuf1, ebuf2, cbuf0, cbuf1, cbuf2,
                   rowt0, rowt1, rowt2, gbuf0, gbuf1, gbuf2, zbuf,
                   se0, se1, se2, sg0, sg1, sg2, ss0, ss1, ss2):
        cid = lax.axis_index("c")
        tid = lax.axis_index("s")
        base = tid * rpt
        zero16 = jnp.zeros((_L,), jnp.float32)
        gbuf = (gbuf0, gbuf1, gbuf2)
        ebuf = (ebuf0, ebuf1, ebuf2)
        cbuf = (cbuf0, cbuf1, cbuf2)
        rowt = (rowt0, rowt1, rowt2)
        se = (se0, se1, se2)
        sg = (sg0, sg1, sg2)
        ss = (ss0, ss1, ss2)

        def zb(i, _):
            for c in range(K // _L):
                zbuf[i, pl.ds(c * _L, _L)] = zero16
            return 0
        lax.fori_loop(0, _L, zb, 0)

        def fill_rowt(p, tbase):
            for k in range(scw // _L):
                rowt[p][pl.ds(k * _L, _L)] = (
                    ebuf[p][0, pl.ds(k * _L, _L)] + tbase)

        def scale_and_cbuf(p):
            def scale(g, _):
                wv = plsc.bitcast(ebuf[p][2, pl.ds(g * _L, _L)], jnp.float32)
                cbuf[p][pl.ds(g * _L, _L)] = ebuf[p][1, pl.ds(g * _L, _L)]
                for l in range(_L):
                    s = wv[l]
                    e = g * _L + l
                    for c in range(K // _L):
                        gbuf[p][e, pl.ds(c * _L, _L)] = (
                            s * gbuf[p][e, pl.ds(c * _L, _L)])
                return 0
            lax.fori_loop(0, scw // _L, scale, 0)

        def eload(p, m):
            pltpu.async_copy(pk_hbm.at[tid, m], ebuf[p], se[p])

        def ewait(p, m):
            pltpu.make_async_copy(pk_hbm.at[tid, m], ebuf[p], se[p]).wait()

        def gstart(p):
            pltpu.async_copy(u_hbm.at[rowt[p]], gbuf[p], sg[p])

        def gwait(p):
            pltpu.make_async_copy(u_hbm.at[rowt[p]], gbuf[p], sg[p]).wait()

        def sstart(p):
            pltpu.async_copy(gbuf[p], sh_v.at[cbuf[p]], ss[p], add=True)

        def swait(p):
            pltpu.make_async_copy(gbuf[p], sh_v.at[cbuf[p]], ss[p]).wait()

        def per_t(tt, _):
            t = cid * t_per_core + tt
            tbase = t * N

            def zv(z, _):
                pltpu.sync_copy(zbuf, sh_v.at[pl.ds(base + z * _L, _L)])
                return 0
            lax.fori_loop(0, rpt // _L, zv, 0)
            plsc.subcore_barrier()

            eload(0, 0)
            eload(1, 1)
            eload(2, 2)
            ewait(0, 0)
            fill_rowt(0, tbase)
            gstart(0)
            ewait(1, 1)
            fill_rowt(1, tbase)
            gstart(1)

            def triple(m, _):
                not_last = m < nc // 3 - 1
                for r in range(3):
                    c3 = 3 * m + r
                    p = r
                    q2 = (r + 2) % 3

                    def prefetch():
                        ewait(q2, c3 + 2)
                        fill_rowt(q2, tbase)

                        @pl.when((m > 0) | (r > 0))
                        def _():
                            swait(q2)
                        gstart(q2)

                    if r == 0:
                        prefetch()
                    else:
                        @pl.when(not_last)
                        def _():
                            prefetch()

                    gwait(p)
                    scale_and_cbuf(p)
                    sstart(p)

                    @pl.when(not_last)
                    def _():
                        eload(p, c3 + 3)
                return 0
            lax.fori_loop(0, nc // 3, triple, 0)
            swait(0)
            swait(1)
            swait(2)
            plsc.subcore_barrier()

            pltpu.sync_copy(sh_v.at[pl.ds(base, rpt)],
                            vp_hbm.at[pl.ds(t * npad + base, rpt)])
            plsc.subcore_barrier()
            return 0
        lax.fori_loop(0, t_per_core, per_t, 0)

    return seg_kernel(pk, U2)


def kernel(x, edge_index, edge_features, W_fl, b_fl, W_el, b_el, att,
           W_cz, b_cz, W_cr, b_cr, W_ch, b_ch, W_lz, b_lz, W_lr, b_lr,
           W_lh, b_lh, W_out, b_out):
    N, F, T = x.shape
    HID = W_cz.shape[1]
    K = 4 * HID

    W_all = jnp.concatenate([W_cz, W_cr, W_ch], axis=1)
    W_comb = jnp.pad(W_fl @ W_all, ((0, 0), (0, HID)))
    b_comb = jnp.pad(b_fl @ W_all, (0, HID))
    bpost = jnp.pad(jnp.concatenate([b_cz, b_cr, b_ch]), (0, HID))
    Wlz1, Wlz2 = W_lz[:HID], W_lz[HID:]
    Wlr1, Wlr2 = W_lr[:HID], W_lr[HID:]
    Wlh1, Wlh2 = W_lh[:HID], W_lh[HID:]
    z = jnp.zeros((HID, HID), jnp.float32)
    Wze = jnp.concatenate([Wlz1, z, z, z], axis=0)
    Wre = jnp.concatenate([z, Wlr1, z, z], axis=0)
    Whe = jnp.concatenate([z, z, Wlh1, z], axis=0)
    probs = jax.nn.softmax(att)

    row = edge_index[0]
    col = edge_index[1]

    ew = _edge_weights(edge_features, W_el, b_el)

    E = row.shape[0]
    nch = -(-E // (_NW * _CW))
    epad = _NW * nch * _CW - E
    npad = ((N + _NT * _L - 1) // (_NT * _L)) * (_NT * _L)
    zi = jnp.zeros((epad,), jnp.int32)
    row32 = jnp.concatenate([row, zi]).reshape(_NW, nch, _CW)
    col32 = jnp.concatenate([col, zi]).reshape(_NW, nch, _CW)
    ew32 = jnp.concatenate([ew, jnp.zeros((epad,), jnp.float32)]).reshape(
        _NW, nch, _CW)

    deg2 = _sc_degree(col32, ew32, npad)
    degp1 = (deg2[0, :N] + deg2[1, :N] + 1.0).reshape(N, 1)

    xT = jnp.transpose(x, (2, 0, 1))
    bn = 1000 if N % 1000 == 0 else N
    U = _project(xT, W_comb, b_comb, degp1, bn)

    U2 = U.reshape(T * N, K)
    scw = 96
    nc = -(-E // (_NT * scw))
    nc = ((nc + 2) // 3) * 3
    epad2 = _NT * nc * scw - E
    zi2 = jnp.zeros((epad2,), jnp.int32)
    row16 = jnp.concatenate([row, zi2]).reshape(_NT, nc, scw)
    col16 = jnp.concatenate([col, zi2]).reshape(_NT, nc, scw)
    ewb16 = lax.bitcast_convert_type(
        jnp.concatenate([ew, jnp.zeros((epad2,), jnp.float32)]),
        jnp.int32).reshape(_NT, nc, scw)
    pk = jnp.stack([row16, col16, ewb16], axis=2)
    Vp_flat = _sc_segment(U2, pk, T, N, K, npad, T // 2)
    Vp = Vp_flat.reshape(T, npad, K)

    return _recurrence(Vp, U, degp1, bpost, Wze, Wre, Whe,
                       Wlz2, Wlr2, Wlh2, b_lz, b_lr, b_lh,
                       probs, W_out, b_out, bn)


# speedup vs baseline: 1.7676x; 1.0060x over previous
import functools
from typing import Any

import jax
import jax.numpy as jnp
import numpy as np
from jax import lax
from jax.experimental import pallas as pl
from jax.experimental.pallas import tpu as pltpu
from jax.experimental.pallas import tpu_sc as plsc


def _ew_body(ef_ref, w_ref, b_ref, out_ref):
    D = ef_ref.shape[0]
    acc = jnp.full((1, ef_ref.shape[1]), b_ref[0, 0], jnp.float32)
    for j in range(D):
        acc = acc + w_ref[j, 0] * ef_ref[j:j + 1, :]
    out_ref[...] = jnp.maximum(acc, 0.0)


def _edge_weights(ef, W_el, b_el):
    E = ef.shape[0]
    D = ef.shape[1]
    efT = jnp.transpose(ef)
    Be = 32000
    grid = (E // Be,)
    out = pl.pallas_call(
        _ew_body,
        grid=grid,
        in_specs=[
            pl.BlockSpec((D, Be), lambda i: (0, i)),
            pl.BlockSpec((D, 1), lambda i: (0, 0)),
            pl.BlockSpec((1, 1), lambda i: (0, 0)),
        ],
        out_specs=pl.BlockSpec((1, Be), lambda i: (0, i)),
        out_shape=jax.ShapeDtypeStruct((1, E), jnp.float32),
    )(efT, W_el, b_el.reshape(1, 1))
    return out[0]


def _proj_body(x_ref, w_ref, b_ref, deg_ref, out_ref):
    dinv = lax.rsqrt(deg_ref[...])
    out_ref[0] = dinv * (
        jnp.dot(x_ref[0], w_ref[...], preferred_element_type=jnp.float32)
        + b_ref[...]
    )


def _project(xT, W_comb, b_comb, degp1, bn):
    T, N, F = xT.shape
    K = W_comb.shape[1]
    grid = (T, N // bn)
    return pl.pallas_call(
        _proj_body,
        grid=grid,
        in_specs=[
            pl.BlockSpec((1, bn, F), lambda t, i: (t, i, 0)),
            pl.BlockSpec((F, K), lambda t, i: (0, 0)),
            pl.BlockSpec((1, K), lambda t, i: (0, 0)),
            pl.BlockSpec((bn, 1), lambda t, i: (i, 0)),
        ],
        out_specs=pl.BlockSpec((1, bn, K), lambda t, i: (t, i, 0)),
        out_shape=jax.ShapeDtypeStruct((T, N, K), jnp.float32),
    )(xT, W_comb, b_comb.reshape(1, K), degp1)


def _recur_body(vp_ref, u_ref, deg_ref, bpost_ref,
                wze_ref, wre_ref, whe_ref,
                wlz2_ref, wlr2_ref, wlh2_ref,
                blz_ref, blr_ref, blh_ref,
                probs_ref, wout_ref, bout_ref, out_ref):
    T = u_ref.shape[0]
    bn = u_ref.shape[1]
    HID = wlz2_ref.shape[0]
    dinv = lax.rsqrt(deg_ref[...])
    H = jnp.zeros((bn, HID), jnp.float32)
    Hacc = jnp.zeros((bn, HID), jnp.float32)
    for t in range(T):
        G = dinv * (vp_ref[t] + u_ref[t]) + bpost_ref[...]
        Z = jax.nn.sigmoid(
            jnp.dot(G, wze_ref[...], preferred_element_type=jnp.float32)
            + jnp.dot(H, wlz2_ref[...], preferred_element_type=jnp.float32)
            + blz_ref[...])
        R = jax.nn.sigmoid(
            jnp.dot(G, wre_ref[...], preferred_element_type=jnp.float32)
            + jnp.dot(H, wlr2_ref[...], preferred_element_type=jnp.float32)
            + blr_ref[...])
        Ht = jnp.tanh(
            jnp.dot(G, whe_ref[...], preferred_element_type=jnp.float32)
            + jnp.dot(H * R, wlh2_ref[...], preferred_element_type=jnp.float32)
            + blh_ref[...])
        H = Z * H + (1.0 - Z) * Ht
        Hacc = Hacc + probs_ref[0, t] * H
    out_ref[...] = (
        jnp.dot(jnp.maximum(Hacc, 0.0), wout_ref[...],
                preferred_element_type=jnp.float32)
        + bout_ref[...])


def _recurrence(Vp, U, degp1, bpost, Wze, Wre, Whe, Wlz2, Wlr2, Wlh2,
                blz, blr, blh, probs, W_out, b_out, bn):
    T, N, K = U.shape
    HID = Wlz2.shape[0]
    TO = W_out.shape[1]
    grid = (N // bn,)
    full = lambda shape: pl.BlockSpec(shape, lambda i: tuple(0 for _ in shape))
    return pl.pallas_call(
        _recur_body,
        grid=grid,
        in_specs=[
            pl.BlockSpec((T, bn, K), lambda i: (0, i, 0)),
            pl.BlockSpec((T, bn, K), lambda i: (0, i, 0)),
            pl.BlockSpec((bn, 1), lambda i: (i, 0)),
            full((1, K)),
            full((K, HID)), full((K, HID)), full((K, HID)),
            full((HID, HID)), full((HID, HID)), full((HID, HID)),
            full((1, HID)), full((1, HID)), full((1, HID)),
            full((1, T)), full((HID, TO)), full((1, TO)),
        ],
        out_specs=pl.BlockSpec((bn, TO), lambda i: (i, 0)),
        out_shape=jax.ShapeDtypeStruct((N, TO), jnp.float32),
    )(Vp, U, degp1, bpost.reshape(1, K),
      Wze, Wre, Whe, Wlz2, Wlr2, Wlh2,
      blz.reshape(1, HID), blr.reshape(1, HID), blh.reshape(1, HID),
      probs.reshape(1, T), W_out, b_out.reshape(1, TO))


_NT = 16
_NW = 32
_CW = 128
_L = 16


def _sc_mesh():
    return plsc.VectorSubcoreMesh(
        core_axis_name="c", subcore_axis_name="s",
        num_cores=2, num_subcores=_NT)


def _sc_degree(col32, ew32, npad):
    nch = col32.shape[1]
    rpt = npad // _NT

    @functools.partial(
        pl.kernel, mesh=_sc_mesh(),
        compiler_params=pltpu.CompilerParams(needs_layout_passes=False),
        out_type=jax.ShapeDtypeStruct((2, npad), jnp.float32),
        scratch_types=[
            pltpu.VMEM_SHARED((npad,), jnp.float32),
            pltpu.VMEM((_CW,), jnp.int32),
            pltpu.VMEM((_CW,), jnp.float32),
            pltpu.VMEM((rpt,), jnp.float32),
        ],
    )
    def deg_kernel(col_hbm, ew_hbm, deg_hbm, sh_deg, cbuf, wbuf, zbuf):
        cid = lax.axis_index("c")
        tid = lax.axis_index("s")
        base = tid * rpt
        zero16 = jnp.zeros((_L,), jnp.float32)

        def zd(i, _):
            zbuf[pl.ds(i * _L, _L)] = zero16
            return 0
        lax.fori_loop(0, rpt // _L, zd, 0)
        pltpu.sync_copy(zbuf, sh_deg.at[pl.ds(base, rpt)])
        plsc.subcore_barrier()

        slab = cid * _NT + tid

        def chunk(j, _):
            pltpu.sync_copy(col_hbm.at[slab, j], cbuf)
            pltpu.sync_copy(ew_hbm.at[slab, j], wbuf)
            pltpu.sync_copy(wbuf, sh_deg.at[cbuf], add=True)
            return 0
        lax.fori_loop(0, nch, chunk, 0)
        plsc.subcore_barrier()

        pltpu.sync_copy(sh_deg.at[pl.ds(base, rpt)],
                        deg_hbm.at[cid, pl.ds(base, rpt)])

    return deg_kernel(col32, ew32)


def _sc_segment(U2, pk, T, N, K, npad, t_per_core):
    nc = pk.shape[1]
    scw = pk.shape[3]
    rpt = npad // _NT
    assert nc % 3 == 0

    @functools.partial(
        pl.kernel, mesh=_sc_mesh(),
        compiler_params=pltpu.CompilerParams(needs_layout_passes=False),
        out_type=jax.ShapeDtypeStruct((T * npad, K), jnp.float32),
        scratch_types=[
            pltpu.VMEM_SHARED((npad, K), jnp.float32),
            pltpu.VMEM((3, scw), jnp.int32),
            pltpu.VMEM((3, scw), jnp.int32),
            pltpu.VMEM((3, scw), jnp.int32),
            pltpu.VMEM((scw,), jnp.int32),
            pltpu.VMEM((scw,), jnp.int32),
            pltpu.VMEM((scw,), jnp.int32),
            pltpu.VMEM((scw,), jnp.int32),
            pltpu.VMEM((scw,), jnp.int32),
            pltpu.VMEM((scw,), jnp.int32),
            pltpu.VMEM((scw, K), jnp.float32),
            pltpu.VMEM((scw, K), jnp.float32),
            pltpu.VMEM((scw, K), jnp.float32),
            pltpu.VMEM((_L, K), jnp.float32),
            pltpu.SemaphoreType.DMA, pltpu.SemaphoreType.DMA,
            pltpu.SemaphoreType.DMA, pltpu.SemaphoreType.DMA,
            pltpu.SemaphoreType.DMA, pltpu.SemaphoreType.DMA,
            pltpu.SemaphoreType.DMA, pltpu.SemaphoreType.DMA,
            pltpu.SemaphoreType.DMA, pltpu.SemaphoreType.DMA,
        ],
    )
    def seg_kernel(pk_hbm, u_hbm, vp_hbm, sh_v,
                   ebuf0, ebuf1, ebuf2, cbuf0, cbuf1, cbuf2,
                   rowt0, rowt1, rowt2, gbuf0, gbuf1, gbuf2, zbuf,
                   se0, se1, se2, sg0, sg1, sg2, ss0, ss1, ss2, sz):
        cid = lax.axis_index("c")
        tid = lax.axis_index("s")
        base = tid * rpt
        zero16 = jnp.zeros((_L,), jnp.float32)
        gbuf = (gbuf0, gbuf1, gbuf2)
        ebuf = (ebuf0, ebuf1, ebuf2)
        cbuf = (cbuf0, cbuf1, cbuf2)
        rowt = (rowt0, rowt1, rowt2)
        se = (se0, se1, se2)
        sg = (sg0, sg1, sg2)
        ss = (ss0, ss1, ss2)

        def zb(i, _):
            for c in range(K // _L):
                zbuf[i, pl.ds(c * _L, _L)] = zero16
            return 0
        lax.fori_loop(0, _L, zb, 0)

        def fill_rowt(p, tbase):
            for k in range(scw // _L):
                rowt[p][pl.ds(k * _L, _L)] = (
                    ebuf[p][0, pl.ds(k * _L, _L)] + tbase)

        def scale_and_cbuf(p):
            def scale(g, _):
                wv = plsc.bitcast(ebuf[p][2, pl.ds(g * _L, _L)], jnp.float32)
                cbuf[p][pl.ds(g * _L, _L)] = ebuf[p][1, pl.ds(g * _L, _L)]
                for l in range(_L):
                    s = wv[l]
                    e = g * _L + l
                    for c in range(K // _L):
                        gbuf[p][e, pl.ds(c * _L, _L)] = (
                            s * gbuf[p][e, pl.ds(c * _L, _L)])
                return 0
            lax.fori_loop(0, scw // _L, scale, 0)

        def eload(p, m):
            pltpu.async_copy(pk_hbm.at[tid, m], ebuf[p], se[p])

        def ewait(p, m):
            pltpu.make_async_copy(pk_hbm.at[tid, m], ebuf[p], se[p]).wait()

        def gstart(p):
            pltpu.async_copy(u_hbm.at[rowt[p]], gbuf[p], sg[p])

        def gwait(p):
            pltpu.make_async_copy(u_hbm.at[rowt[p]], gbuf[p], sg[p]).wait()

        def sstart(p):
            pltpu.async_copy(gbuf[p], sh_v.at[cbuf[p]], ss[p], add=True)

        def swait(p):
            pltpu.make_async_copy(gbuf[p], sh_v.at[cbuf[p]], ss[p]).wait()

        def per_t(tt, _):
            t = cid * t_per_core + tt
            tbase = t * N

            def zv(z, _):
                pltpu.async_copy(zbuf, sh_v.at[pl.ds(base + z * _L, _L)], sz)
                return 0
            lax.fori_loop(0, rpt // _L, zv, 0)

            def zw(z, _):
                pltpu.make_async_copy(
                    zbuf, sh_v.at[pl.ds(base + z * _L, _L)], sz).wait()
                return 0
            lax.fori_loop(0, rpt // _L, zw, 0)
            plsc.subcore_barrier()

            eload(0, 0)
            eload(1, 1)
            eload(2, 2)
            ewait(0, 0)
            fill_rowt(0, tbase)
            gstart(0)
            ewait(1, 1)
            fill_rowt(1, tbase)
            gstart(1)

            def triple(m, _):
                not_last = m < nc // 3 - 1
                for r in range(3):
                    c3 = 3 * m + r
                    p = r
                    q2 = (r + 2) % 3

                    def prefetch():
                        ewait(q2, c3 + 2)
                        fill_rowt(q2, tbase)

                        @pl.when((m > 0) | (r > 0))
                        def _():
                            swait(q2)
                        gstart(q2)

                    if r == 0:
                        prefetch()
                    else:
                        @pl.when(not_last)
                        def _():
                            prefetch()

                    gwait(p)
                    scale_and_cbuf(p)
                    sstart(p)

                    @pl.when(not_last)
                    def _():
                        eload(p, c3 + 3)
                return 0
            lax.fori_loop(0, nc // 3, triple, 0)
            swait(0)
            swait(1)
            swait(2)
            plsc.subcore_barrier()

            pltpu.sync_copy(sh_v.at[pl.ds(base, rpt)],
                            vp_hbm.at[pl.ds(t * npad + base, rpt)])
            plsc.subcore_barrier()
            return 0
        lax.fori_loop(0, t_per_core, per_t, 0)

    return seg_kernel(pk, U2)


def kernel(x, edge_index, edge_features, W_fl, b_fl, W_el, b_el, att,
           W_cz, b_cz, W_cr, b_cr, W_ch, b_ch, W_lz, b_lz, W_lr, b_lr,
           W_lh, b_lh, W_out, b_out):
    N, F, T = x.shape
    HID = W_cz.shape[1]
    K = 4 * HID

    W_all = jnp.concatenate([W_cz, W_cr, W_ch], axis=1)
    W_comb = jnp.pad(W_fl @ W_all, ((0, 0), (0, HID)))
    b_comb = jnp.pad(b_fl @ W_all, (0, HID))
    bpost = jnp.pad(jnp.concatenate([b_cz, b_cr, b_ch]), (0, HID))
    Wlz1, Wlz2 = W_lz[:HID], W_lz[HID:]
    Wlr1, Wlr2 = W_lr[:HID], W_lr[HID:]
    Wlh1, Wlh2 = W_lh[:HID], W_lh[HID:]
    z = jnp.zeros((HID, HID), jnp.float32)
    Wze = jnp.concatenate([Wlz1, z, z, z], axis=0)
    Wre = jnp.concatenate([z, Wlr1, z, z], axis=0)
    Whe = jnp.concatenate([z, z, Wlh1, z], axis=0)
    probs = jax.nn.softmax(att)

    row = edge_index[0]
    col = edge_index[1]

    ew = _edge_weights(edge_features, W_el, b_el)

    E = row.shape[0]
    nch = -(-E // (_NW * _CW))
    epad = _NW * nch * _CW - E
    npad = ((N + _NT * _L - 1) // (_NT * _L)) * (_NT * _L)
    zi = jnp.zeros((epad,), jnp.int32)
    row32 = jnp.concatenate([row, zi]).reshape(_NW, nch, _CW)
    col32 = jnp.concatenate([col, zi]).reshape(_NW, nch, _CW)
    ew32 = jnp.concatenate([ew, jnp.zeros((epad,), jnp.float32)]).reshape(
        _NW, nch, _CW)

    deg2 = _sc_degree(col32, ew32, npad)
    degp1 = (deg2[0, :N] + deg2[1, :N] + 1.0).reshape(N, 1)

    xT = jnp.transpose(x, (2, 0, 1))
    bn = 1000 if N % 1000 == 0 else N
    U = _project(xT, W_comb, b_comb, degp1, bn)

    U2 = U.reshape(T * N, K)
    scw = 96
    nc = -(-E // (_NT * scw))
    nc = ((nc + 2) // 3) * 3
    epad2 = _NT * nc * scw - E
    zi2 = jnp.zeros((epad2,), jnp.int32)
    row16 = jnp.concatenate([row, zi2]).reshape(_NT, nc, scw)
    col16 = jnp.concatenate([col, zi2]).reshape(_NT, nc, scw)
    ewb16 = lax.bitcast_convert_type(
        jnp.concatenate([ew, jnp.zeros((epad2,), jnp.float32)]),
        jnp.int32).reshape(_NT, nc, scw)
    pk = jnp.stack([row16, col16, ewb16], axis=2)
    Vp_flat = _sc_segment(U2, pk, T, N, K, npad, T // 2)
    Vp = Vp_flat.reshape(T, npad, K)

    return _recurrence(Vp, U, degp1, bpost, Wze, Wre, Whe,
                       Wlz2, Wlr2, Wlh2, b_lz, b_lr, b_lh,
                       probs, W_out, b_out, bn)
